# baseline (speedup 1.0000x reference)
import functools

import jax
import jax.numpy as jnp
from jax import lax
from jax.experimental import pallas as pl
from jax.experimental.pallas import tpu as pltpu
from jax.experimental.pallas import tpu_sc as plsc

NUM_CORES = 2
NUM_SUBCORES = 16
NW = NUM_CORES * NUM_SUBCORES
L = 16

BATCH = 16384
D = 64
NUM_ROWS = 1000000
RANGE = 32768
CW = 256
FULL_CHUNKS = RANGE // CW
W30_REG = (999936 - 30 * RANGE) // CW
TAIL_U0 = 999936
DUMMY = BATCH
CAP = 20480
TOTROWS = NW * CAP


def _sweep_body(users_hbm, items_hbm, utabT, itabT, tailTu, tailTi,
                rows_u, rows_i, pb_u, pb_i, cnt_u, cnt_i,
                idxbuf, mu, mb, cu, cb, buf, tbuf, obuf, oent, cbuf, sem, flsem):
    w = lax.axis_index("s") * NUM_CORES + lax.axis_index("c")
    base_w = w * RANGE
    reg_chunks = jnp.where(w == 30, W30_REG,
                           jnp.where(w == 31, 0, FULL_CHUNKS))
    iota = lax.iota(jnp.int32, L)

    for idx_hbm, tabT, tailT, rows_out, pb_out, cnt_out in (
            (users_hbm, utabT, tailTu, rows_u, pb_u, cnt_u),
            (items_hbm, itabT, tailTi, rows_i, pb_i, cnt_i)):
        def scan_piece(p, cnt_p):
            pltpu.sync_copy(
                idx_hbm.at[pl.ds(pl.multiple_of(p * 2048, 8), 2048)], idxbuf)

            def scan(k, cnt):
                b0 = pl.multiple_of(k * L, L)
                u_vec = idxbuf[pl.ds(b0, L)]
                m = (u_vec >> 15) == w
                plsc.store_compressed(mu.at[pl.ds(cnt, L)], u_vec, mask=m)
                plsc.store_compressed(mb.at[pl.ds(cnt, L)],
                                      p * 2048 + b0 + iota, mask=m)
                return cnt + plsc.all_reduce_population_count(m)[0]

            return lax.fori_loop(0, 2048 // L, scan, cnt_p)

        mcnt = lax.fori_loop(0, BATCH // 2048, scan_piece, 0)
        mticks = (mcnt + L - 1) // L

        def chunk_step(c, ocnt):
            is_reg = c < reg_chunks
            is_tail = (w == 30) & (c == W30_REG)
            u0 = pl.multiple_of(base_w + c * CW, 128)

            @pl.when(is_reg)
            def _():
                cps = [pltpu.async_copy(
                    tabT.at[pl.ds(dh * 8, 8), pl.ds(u0, CW)],
                    buf.at[dh], sem) for dh in range(8)]
                for cp in cps:
                    cp.wait()

            @pl.when(is_tail)
            def _():
                cps = [pltpu.async_copy(
                    tailT.at[pl.ds(dh * 8, 8), :],
                    buf.at[dh, :, pl.ds(0, 128)], sem) for dh in range(8)]
                for cp in cps:
                    cp.wait()

            def collect(j, ccnt):
                p0 = pl.multiple_of(j * L, L)
                u_vec = mu[pl.ds(p0, L)]
                b_vec = mb[pl.ds(p0, L)]
                m = ((p0 + iota) < mcnt) & (((u_vec - base_w) >> 8) == c)
                plsc.store_compressed(cu.at[pl.ds(ccnt, L)], u_vec, mask=m)
                plsc.store_compressed(cb.at[pl.ds(ccnt, L)], b_vec, mask=m)
                return ccnt + plsc.all_reduce_population_count(m)[0]

            ccnt = lax.fori_loop(0, mticks, collect, 0)
            cticks = (ccnt + L - 1) // L

            def batch(e, ocnt_in):
                p0 = pl.multiple_of(e * L, L)
                u_vec = cu[pl.ds(p0, L)]
                b_vec = cb[pl.ds(p0, L)]
                vmask = (p0 + iota) < ccnt
                uloc = (u_vec - u0) & (CW - 1)
                for q in range(D):
                    g = plsc.load_gather(
                        buf, [jnp.full((L,), q >> 3, jnp.int32),
                              jnp.full((L,), q & 7, jnp.int32), uloc],
                        mask=vmask)
                    tbuf[q] = g
                half = (ocnt_in >> 7) & 1
                om = pl.multiple_of(ocnt_in & 127, L)

                @pl.when((ocnt_in & 127) == 0)
                def _():
                    @pl.when(ocnt_in >= 256)
                    def _():
                        pltpu.make_async_copy(
                            rows_out.at[pl.ds(0, 128), :], obuf.at[half],
                            flsem).wait()
                        pltpu.make_async_copy(
                            pb_out.at[pl.ds(0, 128)], oent.at[half],
                            flsem).wait()

                for l in range(L):
                    for qq in range(D // L):
                        r = plsc.load_gather(
                            tbuf, [qq * L + iota, jnp.full((L,), l, jnp.int32)])
                        obuf[half, om + l, pl.ds(qq * L, L)] = r
                oent[half, pl.ds(om, L)] = jnp.where(vmask, b_vec, DUMMY)

                @pl.when((ocnt_in & 127) == 112)
                def _():
                    g0 = pl.multiple_of(w * CAP + (ocnt_in - 112), 128)
                    pltpu.async_copy(obuf.at[half],
                                     rows_out.at[pl.ds(g0, 128), :], flsem)
                    pltpu.async_copy(oent.at[half],
                                     pb_out.at[pl.ds(g0, 128)], flsem)

                return ocnt_in + L

            return lax.fori_loop(0, cticks, batch, ocnt)

        ocnt = lax.fori_loop(0, FULL_CHUNKS + 1, chunk_step, 0)
        nfl = ocnt >> 7

        def drain(_k, _c):
            pltpu.make_async_copy(rows_out.at[pl.ds(0, 128), :],
                                  obuf.at[0], flsem).wait()
            pltpu.make_async_copy(pb_out.at[pl.ds(0, 128)],
                                  oent.at[0], flsem).wait()
            return 0

        outst = jnp.where((ocnt & 127) != 0, jnp.minimum(nfl, 1),
                          jnp.minimum(nfl, 2))
        lax.fori_loop(0, outst, drain, 0)
        half = (ocnt >> 7) & 1
        g0 = pl.multiple_of(w * CAP + (ocnt & ~jnp.int32(127)), 128)
        pltpu.sync_copy(obuf.at[half], rows_out.at[pl.ds(g0, 128), :])
        pltpu.sync_copy(oent.at[half], pb_out.at[pl.ds(g0, 128)])
        for j in range(4):
            cbuf[pl.ds(j * L, L)] = jnp.broadcast_to(ocnt, (L,))
        pltpu.sync_copy(cbuf, cnt_out.at[pl.ds(w * 64, 64)])


def _dot_body(rows_u, rows_i, pb_u, pb_i, cnt_u, cnt_i, out_hbm,
              cbuf, pbbuf, pos_u, pos_i, bu, bi, scratch, out_v, sem):
    w = lax.axis_index("s") * NUM_CORES + lax.axis_index("c")
    iota = lax.iota(jnp.int32, L)
    iota16 = iota * L

    for cnt_in, pb_in, pos in ((cnt_u, pb_u, pos_u), (cnt_i, pb_i, pos_i)):
        pltpu.sync_copy(cnt_in, cbuf)

        for v in range(NW):
            cnt_v = cbuf[pl.ds(v * 64, L)][0]
            npieces = (cnt_v + 2047) // 2048

            def piece(p, _):
                pltpu.sync_copy(
                    pb_in.at[pl.ds(pl.multiple_of(v * CAP + p * 2048, 8),
                                   2048)], pbbuf)

                def scan(j, _2):
                    b_vec = pbbuf[pl.ds(pl.multiple_of(j * L, L), L)]
                    e = p * 2048 + j * L + iota
                    m = (e < cnt_v) & ((b_vec >> 9) == w)
                    bl = b_vec & 511
                    plsc.store_scatter(pos, [bl >> 7, bl & 127],
                                       v * CAP + e, mask=m)
                    return 0

                jticks = jnp.minimum(2048 // L,
                                     (cnt_v - p * 2048 + L - 1) // L)
                lax.fori_loop(0, jticks, scan, 0)
                return 0

            lax.fori_loop(0, npieces, piece, 0)

    def sub(s, _):
        cpu = pltpu.async_copy(rows_u.at[pos_u.at[s]], bu, sem)
        cpi = pltpu.async_copy(rows_i.at[pos_i.at[s]], bi, sem)
        cpu.wait()
        cpi.wait()

        def group(g, _2):
            for k in range(L):
                r = g * L + k
                acc = bu[r, pl.ds(0, L)] * bi[r, pl.ds(0, L)]
                for c in range(1, D // L):
                    acc = acc + (bu[r, pl.ds(c * L, L)]
                                 * bi[r, pl.ds(c * L, L)])
                scratch[pl.ds(k * L, L)] = acc
            res = plsc.load_gather(scratch, [iota16])
            for j in range(1, L):
                res = res + plsc.load_gather(scratch, [iota16 + j])
            out_v[pl.ds(pl.multiple_of(s * 128 + g * L, L), L)] = res
            return 0

        lax.fori_loop(0, 8, group, 0)
        return 0

    lax.fori_loop(0, 4, sub, 0)
    pltpu.sync_copy(out_v, out_hbm.at[pl.ds(w * 512, 512)])


@jax.jit
def _bpr_sc(users, items, user_table, item_table):
    utabT = user_table.T
    itabT = item_table.T
    pad = ((0, 0), (0, 128 - (NUM_ROWS - TAIL_U0)))
    tailTu = jnp.pad(utabT[:, TAIL_U0:], pad)
    tailTi = jnp.pad(itabT[:, TAIL_U0:], pad)

    mesh = plsc.VectorSubcoreMesh(
        core_axis_name="c", subcore_axis_name="s",
        num_cores=NUM_CORES, num_subcores=NUM_SUBCORES)

    rows_u, rows_i, pb_u, pb_i, cnt_u, cnt_i = pl.kernel(
        _sweep_body,
        out_type=(jax.ShapeDtypeStruct((TOTROWS, D), jnp.float32),
                  jax.ShapeDtypeStruct((TOTROWS, D), jnp.float32),
                  jax.ShapeDtypeStruct((TOTROWS,), jnp.int32),
                  jax.ShapeDtypeStruct((TOTROWS,), jnp.int32),
                  jax.ShapeDtypeStruct((NW * 64,), jnp.int32),
                  jax.ShapeDtypeStruct((NW * 64,), jnp.int32)),
        mesh=mesh,
        compiler_params=pltpu.CompilerParams(
            needs_layout_passes=False, use_tc_tiling_on_sc=True),
        scratch_types=[
            pltpu.VMEM((2048,), jnp.int32),
            pltpu.VMEM((BATCH,), jnp.int32),
            pltpu.VMEM((BATCH,), jnp.int32),
            pltpu.VMEM((BATCH,), jnp.int32),
            pltpu.VMEM((BATCH,), jnp.int32),
            pltpu.VMEM((8, 8, CW), jnp.float32),
            pltpu.VMEM((D, L), jnp.float32),
            pltpu.VMEM((2, 128, D), jnp.float32),
            pltpu.VMEM((2, 128), jnp.int32),
            pltpu.VMEM((64,), jnp.int32),
            pltpu.SemaphoreType.DMA,
            pltpu.SemaphoreType.DMA,
        ],
    )(users, items, utabT, itabT, tailTu, tailTi)

    return pl.kernel(
        _dot_body,
        out_type=jax.ShapeDtypeStruct((BATCH,), jnp.float32),
        mesh=mesh,
        compiler_params=pltpu.CompilerParams(
            needs_layout_passes=False, use_tc_tiling_on_sc=False),
        scratch_types=[
            pltpu.VMEM((NW * 64,), jnp.int32),
            pltpu.VMEM((2048,), jnp.int32),
            pltpu.VMEM((4, 128), jnp.int32),
            pltpu.VMEM((4, 128), jnp.int32),
            pltpu.VMEM((128, D), jnp.float32),
            pltpu.VMEM((128, D), jnp.float32),
            pltpu.VMEM((L * L,), jnp.float32),
            pltpu.VMEM((512,), jnp.float32),
            pltpu.SemaphoreType.DMA,
        ],
    )(rows_u, rows_i, pb_u, pb_i, cnt_u, cnt_i)


def kernel(users, items, user_table, item_table):
    return _bpr_sc(users.astype(jnp.int32), items.astype(jnp.int32),
                   user_table, item_table)


# speedup vs baseline: 1.2541x; 1.2541x over previous
import functools

import jax
import jax.numpy as jnp
from jax import lax
from jax.experimental import pallas as pl
from jax.experimental.pallas import tpu as pltpu
from jax.experimental.pallas import tpu_sc as plsc

NUM_CORES = 2
NUM_SUBCORES = 16
NUM_WORKERS = NUM_CORES * NUM_SUBCORES
LANES = 16

BATCH = 16384
EMBED_DIM = 64
ROWS_PER_WORKER = BATCH // NUM_WORKERS
CHUNK = 128
NUM_CHUNKS = ROWS_PER_WORKER // CHUNK
GROUPS = ROWS_PER_WORKER // LANES


def _bpr_body(users_hbm, items_hbm, utab_hbm, itab_hbm, out_hbm,
              idx_u, idx_i, rows_u, rows_i, out_v, scratch_flat, sem):
    wid = lax.axis_index("s") * NUM_CORES + lax.axis_index("c")

    base = wid * NUM_CHUNKS
    pltpu.sync_copy(users_hbm.at[pl.ds(base, NUM_CHUNKS)], idx_u)
    pltpu.sync_copy(items_hbm.at[pl.ds(base, NUM_CHUNKS)], idx_i)

    copies = []
    for c in range(NUM_CHUNKS):
        copies.append(pltpu.async_copy(
            utab_hbm.at[idx_u.at[c]], rows_u.at[pl.ds(c * CHUNK, CHUNK)], sem))
        copies.append(pltpu.async_copy(
            itab_hbm.at[idx_i.at[c]], rows_i.at[pl.ds(c * CHUNK, CHUNK)], sem))
    for cp in copies:
        cp.wait()

    iota = lax.iota(jnp.int32, LANES)

    def group(g, _):
        for k in range(LANES):
            r = g * LANES + k
            acc = rows_u[r, pl.ds(0, LANES)] * rows_i[r, pl.ds(0, LANES)]
            for c in range(1, EMBED_DIM // LANES):
                acc = acc + (rows_u[r, pl.ds(c * LANES, LANES)]
                             * rows_i[r, pl.ds(c * LANES, LANES)])
            scratch_flat[pl.ds(k * LANES, LANES)] = acc
        iota16 = iota * LANES
        res = plsc.load_gather(scratch_flat, [iota16])
        for j in range(1, LANES):
            res = res + plsc.load_gather(scratch_flat, [iota16 + j])
        out_v[pl.ds(pl.multiple_of(g * LANES, LANES), LANES)] = res
        return 0

    lax.fori_loop(0, GROUPS, group, 0)

    pltpu.sync_copy(out_v, out_hbm.at[pl.ds(wid * ROWS_PER_WORKER,
                                            ROWS_PER_WORKER)])


@jax.jit
def _bpr_sc(users2d, items2d, user_table, item_table):
    mesh = plsc.VectorSubcoreMesh(
        core_axis_name="c", subcore_axis_name="s",
        num_cores=NUM_CORES, num_subcores=NUM_SUBCORES)
    return pl.kernel(
        _bpr_body,
        out_type=jax.ShapeDtypeStruct((BATCH,), jnp.float32),
        mesh=mesh,
        compiler_params=pltpu.CompilerParams(
            needs_layout_passes=False, use_tc_tiling_on_sc=False),
        scratch_types=[
            pltpu.VMEM((NUM_CHUNKS, CHUNK), jnp.int32),
            pltpu.VMEM((NUM_CHUNKS, CHUNK), jnp.int32),
            pltpu.VMEM((ROWS_PER_WORKER, EMBED_DIM), jnp.float32),
            pltpu.VMEM((ROWS_PER_WORKER, EMBED_DIM), jnp.float32),
            pltpu.VMEM((ROWS_PER_WORKER,), jnp.float32),
            pltpu.VMEM((LANES * LANES,), jnp.float32),
            pltpu.SemaphoreType.DMA,
        ],
    )(users2d, items2d, user_table, item_table)


def kernel(users, items, user_table, item_table):
    users2d = users.astype(jnp.int32).reshape(NUM_WORKERS * NUM_CHUNKS, CHUNK)
    items2d = items.astype(jnp.int32).reshape(NUM_WORKERS * NUM_CHUNKS, CHUNK)
    return _bpr_sc(users2d, items2d, user_table, item_table)
